# baseline (speedup 1.0000x reference)
import jax
import jax.numpy as jnp
from jax import lax
from jax.experimental import pallas as pl
from jax.experimental.pallas import tpu as pltpu
from jax.experimental.pallas import tpu_sc as plsc

B = 4
HO = WO = 192
CC = 192
OUT_B = HO * WO * CC
IN_B = OUT_B // 4
TOTAL_OUT = B * OUT_B
NS = 16
NWIN = 2
WIN = OUT_B // (2 * NWIN)
SHARE = IN_B // NS
WSHARE = WIN // NS
CHUNK = 4608
NCHUNK = SHARE // CHUNK
GROUPS = CHUNK // 16
PK = 2 * CHUNK


def _scatter_body(pkd_hbm, z_hbm, out_hbm, win_sh, pk, off_v, val_v, zsem):
    c = lax.axis_index("c")
    s = lax.axis_index("s")

    def load_src(b, ch):
        return pkd_hbm.at[pl.ds(2 * (b * IN_B + s * SHARE) + ch * PK, PK)]

    def compute_chunk(wbase):
        def body(g, carry):
            iv = pk[pl.ds(g * 16, 16)]
            uv = lax.bitcast_convert_type(pk[pl.ds(CHUNK + g * 16, 16)],
                                          jnp.float32)
            rel = iv - wbase
            inm = (iv >= wbase) & (rel < WIN)
            off_v[pl.ds(g * 16, 16)] = jnp.where(inm, rel, iv & 0xFFFF)
            val_v[pl.ds(g * 16, 16)] = jnp.where(
                inm, uv, jnp.zeros((16,), jnp.float32))
            return carry

        lax.fori_loop(0, GROUPS, body, 0)

    for b in range(B):
        for w in range(NWIN):
            wbase = c * (NWIN * WIN) + w * WIN

            zdma = pltpu.async_copy(
                z_hbm.at[pl.ds(s * WSHARE, WSHARE)],
                win_sh.at[pl.ds(s * WSHARE, WSHARE)], zsem)
            pltpu.sync_copy(load_src(b, 0), pk)
            compute_chunk(wbase)
            zdma.wait()
            plsc.subcore_barrier()

            for ch in range(NCHUNK):
                pltpu.sync_copy(val_v, win_sh.at[off_v], add=True)
                if ch + 1 < NCHUNK:
                    pltpu.sync_copy(load_src(b, ch + 1), pk)
                    compute_chunk(wbase)
            plsc.subcore_barrier()

            out_base = b * OUT_B + wbase + s * WSHARE
            pltpu.sync_copy(win_sh.at[pl.ds(s * WSHARE, WSHARE)],
                            out_hbm.at[pl.ds(out_base, WSHARE)])
            plsc.subcore_barrier()


def kernel(updates, mask):
    idx = mask.reshape(-1).astype(jnp.int32)
    upd = jax.lax.bitcast_convert_type(updates.reshape(-1), jnp.int32)
    packed = jnp.stack(
        [idx.reshape(-1, CHUNK), upd.reshape(-1, CHUNK)], axis=1).reshape(-1)
    zeros = jnp.zeros((WIN,), jnp.float32)
    mesh = plsc.VectorSubcoreMesh(core_axis_name="c", subcore_axis_name="s")
    run = pl.kernel(
        _scatter_body,
        mesh=mesh,
        out_type=jax.ShapeDtypeStruct((TOTAL_OUT,), jnp.float32),
        scratch_types=[
            pltpu.VMEM_SHARED((WIN,), jnp.float32),
            pltpu.VMEM((PK,), jnp.int32),
            pltpu.VMEM((CHUNK,), jnp.int32),
            pltpu.VMEM((CHUNK,), jnp.float32),
            pltpu.SemaphoreType.DMA,
        ],
    )
    out = run(packed, zeros)
    return out.reshape(B, HO, WO, CC)


# speedup vs baseline: 1.7670x; 1.7670x over previous
import jax
import jax.numpy as jnp
from jax import lax
from jax.experimental import pallas as pl
from jax.experimental.pallas import tpu as pltpu
from jax.experimental.pallas import tpu_sc as plsc

B = 4
HO = WO = 192
CC = 192
OUT_B = HO * WO * CC
IN_B = OUT_B // 4
TOTAL_OUT = B * OUT_B
NS = 16
NWIN = 2
WIN = OUT_B // (2 * NWIN)
SHARE = IN_B // NS
WSHARE = WIN // NS
CHUNK = 4608
NCHUNK = SHARE // CHUNK
GROUPS = CHUNK // 16


def _scatter_body(idx_hbm, upd_hbm, out_hbm, win_sh, idx_v, upd_v, off_v,
                  val_v, ssem):
    c = lax.axis_index("c")
    s = lax.axis_index("s")

    def load_chunk(b, ch):
        base = b * IN_B + s * SHARE + ch * CHUNK
        pltpu.sync_copy(idx_hbm.at[pl.ds(base, CHUNK)], idx_v)
        pltpu.sync_copy(upd_hbm.at[pl.ds(base, CHUNK)], upd_v)

    def compute_chunk(wbase):
        def body(g, carry):
            iv = idx_v[pl.ds(g * 16, 16)]
            uv = upd_v[pl.ds(g * 16, 16)]
            rel = iv - wbase
            inm = (iv >= wbase) & (rel < WIN)
            off_v[pl.ds(g * 16, 16)] = jnp.where(inm, rel, iv & 0xFFFF)
            val_v[pl.ds(g * 16, 16)] = jnp.where(
                inm, uv, jnp.zeros((16,), jnp.float32))
            return carry

        lax.fori_loop(0, GROUPS, body, 0)

    for b in range(B):
        for w in range(NWIN):
            wbase = c * (NWIN * WIN) + w * WIN

            def zfill(g, carry):
                val_v[pl.ds(g * 16, 16)] = jnp.zeros((16,), jnp.float32)
                return carry

            lax.fori_loop(0, GROUPS, zfill, 0)
            for z in range(WSHARE // CHUNK):
                pltpu.sync_copy(
                    val_v, win_sh.at[pl.ds(s * WSHARE + z * CHUNK, CHUNK)])
            plsc.subcore_barrier()

            load_chunk(b, 0)
            compute_chunk(wbase)
            for ch in range(NCHUNK):
                sdma = pltpu.async_copy(val_v, win_sh.at[off_v], ssem,
                                        add=True)
                if ch + 1 < NCHUNK:
                    load_chunk(b, ch + 1)
                sdma.wait()
                if ch + 1 < NCHUNK:
                    compute_chunk(wbase)
            plsc.subcore_barrier()

            out_base = b * OUT_B + wbase + s * WSHARE
            pltpu.sync_copy(win_sh.at[pl.ds(s * WSHARE, WSHARE)],
                            out_hbm.at[pl.ds(out_base, WSHARE)])
            plsc.subcore_barrier()


def kernel(updates, mask):
    idx = mask.reshape(-1).astype(jnp.int32)
    upd = updates.reshape(-1)
    mesh = plsc.VectorSubcoreMesh(core_axis_name="c", subcore_axis_name="s")
    run = pl.kernel(
        _scatter_body,
        mesh=mesh,
        out_type=jax.ShapeDtypeStruct((TOTAL_OUT,), jnp.float32),
        scratch_types=[
            pltpu.VMEM_SHARED((WIN,), jnp.float32),
            pltpu.VMEM((CHUNK,), jnp.int32),
            pltpu.VMEM((CHUNK,), jnp.float32),
            pltpu.VMEM((CHUNK,), jnp.int32),
            pltpu.VMEM((CHUNK,), jnp.float32),
            pltpu.SemaphoreType.DMA,
        ],
    )
    out = run(idx, upd)
    return out.reshape(B, HO, WO, CC)


# speedup vs baseline: 1.8522x; 1.0482x over previous
import jax
import jax.numpy as jnp
from jax import lax
from jax.experimental import pallas as pl
from jax.experimental.pallas import tpu as pltpu
from jax.experimental.pallas import tpu_sc as plsc

B = 4
HO = WO = 192
CC = 192
OUT_B = HO * WO * CC
IN_B = OUT_B // 4
TOTAL_OUT = B * OUT_B
NS = 16
NWIN = 2
WIN = OUT_B // (2 * NWIN)
SHARE = IN_B // NS
WSHARE = WIN // NS
CHUNK = 4608
NCHUNK = SHARE // CHUNK
GROUPS = CHUNK // 16


def _scatter_body(idx_hbm, upd_hbm, z_hbm, out_hbm, win_sh, idx_v, upd_v,
                  off_v, val_v, ssem, l1sem, l2sem, zsem):
    c = lax.axis_index("c")
    s = lax.axis_index("s")

    def load_chunk(b, ch):
        base = b * IN_B + s * SHARE + ch * CHUNK
        d1 = pltpu.async_copy(idx_hbm.at[pl.ds(base, CHUNK)], idx_v, l1sem)
        d2 = pltpu.async_copy(upd_hbm.at[pl.ds(base, CHUNK)], upd_v, l2sem)
        d1.wait()
        d2.wait()

    def compute_chunk(wbase):
        def body(g, carry):
            iv = idx_v[pl.ds(g * 16, 16)]
            uv = upd_v[pl.ds(g * 16, 16)]
            rel = iv - wbase
            inm = (iv >= wbase) & (rel < WIN)
            off_v[pl.ds(g * 16, 16)] = jnp.where(inm, rel, iv & 0xFFFF)
            val_v[pl.ds(g * 16, 16)] = jnp.where(
                inm, uv, jnp.zeros((16,), jnp.float32))
            return carry

        lax.fori_loop(0, GROUPS, body, 0)

    for b in range(B):
        for w in range(NWIN):
            wbase = c * (NWIN * WIN) + w * WIN

            zdma = pltpu.async_copy(
                z_hbm.at[pl.ds(s * WSHARE, WSHARE)],
                win_sh.at[pl.ds(s * WSHARE, WSHARE)], zsem)
            load_chunk(b, 0)
            compute_chunk(wbase)
            zdma.wait()
            plsc.subcore_barrier()

            for ch in range(NCHUNK):
                sdma = pltpu.async_copy(val_v, win_sh.at[off_v], ssem,
                                        add=True)
                if ch + 1 < NCHUNK:
                    load_chunk(b, ch + 1)
                sdma.wait()
                if ch + 1 < NCHUNK:
                    compute_chunk(wbase)
            plsc.subcore_barrier()

            out_base = b * OUT_B + wbase + s * WSHARE
            pltpu.sync_copy(win_sh.at[pl.ds(s * WSHARE, WSHARE)],
                            out_hbm.at[pl.ds(out_base, WSHARE)])
            plsc.subcore_barrier()


def kernel(updates, mask):
    idx = mask.reshape(-1).astype(jnp.int32)
    upd = updates.reshape(-1)
    zeros = jnp.zeros((WIN,), jnp.float32)
    mesh = plsc.VectorSubcoreMesh(core_axis_name="c", subcore_axis_name="s")
    run = pl.kernel(
        _scatter_body,
        mesh=mesh,
        out_type=jax.ShapeDtypeStruct((TOTAL_OUT,), jnp.float32),
        scratch_types=[
            pltpu.VMEM_SHARED((WIN,), jnp.float32),
            pltpu.VMEM((CHUNK,), jnp.int32),
            pltpu.VMEM((CHUNK,), jnp.float32),
            pltpu.VMEM((CHUNK,), jnp.int32),
            pltpu.VMEM((CHUNK,), jnp.float32),
            pltpu.SemaphoreType.DMA,
            pltpu.SemaphoreType.DMA,
            pltpu.SemaphoreType.DMA,
            pltpu.SemaphoreType.DMA,
        ],
    )
    out = run(idx, upd, zeros)
    return out.reshape(B, HO, WO, CC)


# speedup vs baseline: 1.9442x; 1.0497x over previous
import jax
import jax.numpy as jnp
from jax import lax
from jax.experimental import pallas as pl
from jax.experimental.pallas import tpu as pltpu
from jax.experimental.pallas import tpu_sc as plsc

B = 4
HO = WO = 192
CC = 192
OUT_B = HO * WO * CC
IN_B = OUT_B // 4
TOTAL_OUT = B * OUT_B
NS = 16
NWIN = 3
WIN = OUT_B // (2 * NWIN)
SHARE = IN_B // NS
WSHARE = WIN // NS
CHUNK = 6912
NCHUNK = SHARE // CHUNK
GROUPS = CHUNK // 16


def _scatter_body(idx_hbm, upd_hbm, z_hbm, out_hbm, win_sh,
                  idx0, idx1, upd0, upd1, off0, off1, val0, val1,
                  l1s0, l1s1, l2s0, l2s1, ss0, ss1, zsem):
    c = lax.axis_index("c")
    s = lax.axis_index("s")
    idx_v = (idx0, idx1)
    upd_v = (upd0, upd1)
    off_v = (off0, off1)
    val_v = (val0, val1)
    l1sem = (l1s0, l1s1)
    l2sem = (l2s0, l2s1)
    ssem = (ss0, ss1)

    def issue_loads(b, ch, j):
        base = b * IN_B + s * SHARE + ch * CHUNK
        d1 = pltpu.async_copy(idx_hbm.at[pl.ds(base, CHUNK)], idx_v[j],
                              l1sem[j])
        d2 = pltpu.async_copy(upd_hbm.at[pl.ds(base, CHUNK)], upd_v[j],
                              l2sem[j])
        return (d1, d2)

    def compute_chunk(wbase, j):
        def body(g, carry):
            iv = idx_v[j][pl.ds(g * 16, 16)]
            uv = upd_v[j][pl.ds(g * 16, 16)]
            rel = iv - wbase
            inm = (iv >= wbase) & (rel < WIN)
            off_v[j][pl.ds(g * 16, 16)] = jnp.where(inm, rel, iv & 0xFFFF)
            val_v[j][pl.ds(g * 16, 16)] = jnp.where(
                inm, uv, jnp.zeros((16,), jnp.float32))
            return carry

        lax.fori_loop(0, GROUPS, body, 0)

    for b in range(B):
        for w in range(NWIN):
            wbase = c * (NWIN * WIN) + w * WIN

            zdma = pltpu.async_copy(
                z_hbm.at[pl.ds(s * WSHARE, WSHARE)],
                win_sh.at[pl.ds(s * WSHARE, WSHARE)], zsem)
            loads = [issue_loads(b, 0, 0), issue_loads(b, 1, 1)]
            zdma.wait()
            plsc.subcore_barrier()

            sdmas = [None, None]
            for ch in range(NCHUNK):
                j = ch % 2
                loads[j][0].wait()
                loads[j][1].wait()
                if sdmas[j] is not None:
                    sdmas[j].wait()
                compute_chunk(wbase, j)
                sdmas[j] = pltpu.async_copy(val_v[j], win_sh.at[off_v[j]],
                                            ssem[j], add=True)
                if ch + 2 < NCHUNK:
                    loads[j] = issue_loads(b, ch + 2, j)
            sdmas[0].wait()
            sdmas[1].wait()
            plsc.subcore_barrier()

            out_base = b * OUT_B + wbase + s * WSHARE
            pltpu.sync_copy(win_sh.at[pl.ds(s * WSHARE, WSHARE)],
                            out_hbm.at[pl.ds(out_base, WSHARE)])
            plsc.subcore_barrier()


def kernel(updates, mask):
    idx = mask.reshape(-1).astype(jnp.int32)
    upd = updates.reshape(-1)
    zeros = jnp.zeros((WIN,), jnp.float32)
    mesh = plsc.VectorSubcoreMesh(core_axis_name="c", subcore_axis_name="s")
    run = pl.kernel(
        _scatter_body,
        mesh=mesh,
        out_type=jax.ShapeDtypeStruct((TOTAL_OUT,), jnp.float32),
        scratch_types=[
            pltpu.VMEM_SHARED((WIN,), jnp.float32),
            pltpu.VMEM((CHUNK,), jnp.int32),
            pltpu.VMEM((CHUNK,), jnp.int32),
            pltpu.VMEM((CHUNK,), jnp.float32),
            pltpu.VMEM((CHUNK,), jnp.float32),
            pltpu.VMEM((CHUNK,), jnp.int32),
            pltpu.VMEM((CHUNK,), jnp.int32),
            pltpu.VMEM((CHUNK,), jnp.float32),
            pltpu.VMEM((CHUNK,), jnp.float32),
            pltpu.SemaphoreType.DMA,
            pltpu.SemaphoreType.DMA,
            pltpu.SemaphoreType.DMA,
            pltpu.SemaphoreType.DMA,
            pltpu.SemaphoreType.DMA,
            pltpu.SemaphoreType.DMA,
            pltpu.SemaphoreType.DMA,
        ],
    )
    out = run(idx, upd, zeros)
    return out.reshape(B, HO, WO, CC)


# speedup vs baseline: 2.0565x; 1.0578x over previous
import jax
import jax.numpy as jnp
from jax import lax
from jax.experimental import pallas as pl
from jax.experimental.pallas import tpu as pltpu
from jax.experimental.pallas import tpu_sc as plsc

B = 4
HO = WO = 192
CC = 192
OUT_B = HO * WO * CC
IN_B = OUT_B // 4
TOTAL_OUT = B * OUT_B
NS = 16
NWIN = 3
WIN = OUT_B // (2 * NWIN)
SHARE = IN_B // NS
WSHARE = WIN // NS
CHUNK = 6912
NCHUNK = SHARE // CHUNK
GROUPS = CHUNK // 16


def _scatter_body(idx_hbm, upd_hbm, z_hbm, out_hbm, win_sh,
                  idx0, idx1, upd0, upd1, off0, off1, val0, val1,
                  l1s0, l1s1, l2s0, l2s1, ss0, ss1, zsem):
    c = lax.axis_index("c")
    s = lax.axis_index("s")
    idx_v = (idx0, idx1)
    upd_v = (upd0, upd1)
    off_v = (off0, off1)
    val_v = (val0, val1)
    l1sem = (l1s0, l1s1)
    l2sem = (l2s0, l2s1)
    ssem = (ss0, ss1)

    def issue_loads(b, ch, j):
        base = b * IN_B + s * SHARE + ch * CHUNK
        d1 = pltpu.async_copy(idx_hbm.at[pl.ds(base, CHUNK)], idx_v[j],
                              l1sem[j])
        d2 = pltpu.async_copy(upd_hbm.at[pl.ds(base, CHUNK)], upd_v[j],
                              l2sem[j])
        return (d1, d2)

    def compute_chunk(wbase, j):
        def body(g, carry):
            iv = idx_v[j][pl.ds(g * 16, 16)]
            uv = upd_v[j][pl.ds(g * 16, 16)]
            rel = iv - wbase
            inm = (iv >= wbase) & (rel < WIN)
            off_v[j][pl.ds(g * 16, 16)] = jnp.where(inm, rel, iv & 0xFFFF)
            val_v[j][pl.ds(g * 16, 16)] = jnp.where(
                inm, uv, jnp.zeros((16,), jnp.float32))
            return carry

        lax.fori_loop(0, GROUPS, body, 0)

    for b in range(B):
        for w in range(NWIN):
            wbase = c * (NWIN * WIN) + w * WIN

            zdma = pltpu.async_copy(
                z_hbm.at[pl.ds(s * WSHARE, WSHARE)],
                win_sh.at[pl.ds(s * WSHARE, WSHARE)], zsem)
            loads = [issue_loads(b, 0, 0), issue_loads(b, 1, 1)]
            zdma.wait()
            plsc.subcore_barrier()

            sdmas = [None, None]
            for ch in range(NCHUNK):
                j = ch % 2
                loads[j][0].wait()
                loads[j][1].wait()
                if sdmas[j] is not None:
                    sdmas[j].wait()
                compute_chunk(wbase, j)
                sdmas[j] = pltpu.async_copy(val_v[j], win_sh.at[off_v[j]],
                                            ssem[j], add=True)
                if ch + 2 < NCHUNK:
                    loads[j] = issue_loads(b, ch + 2, j)
            sdmas[0].wait()
            sdmas[1].wait()
            plsc.subcore_barrier()

            out_base = b * OUT_B + wbase + s * WSHARE
            pltpu.sync_copy(win_sh.at[pl.ds(s * WSHARE, WSHARE)],
                            out_hbm.at[pl.ds(out_base, WSHARE)])


def kernel(updates, mask):
    idx = mask.reshape(-1).astype(jnp.int32)
    upd = updates.reshape(-1)
    zeros = jnp.zeros((WIN,), jnp.float32)
    mesh = plsc.VectorSubcoreMesh(core_axis_name="c", subcore_axis_name="s")
    run = pl.kernel(
        _scatter_body,
        mesh=mesh,
        out_type=jax.ShapeDtypeStruct((TOTAL_OUT,), jnp.float32),
        scratch_types=[
            pltpu.VMEM_SHARED((WIN,), jnp.float32),
            pltpu.VMEM((CHUNK,), jnp.int32),
            pltpu.VMEM((CHUNK,), jnp.int32),
            pltpu.VMEM((CHUNK,), jnp.float32),
            pltpu.VMEM((CHUNK,), jnp.float32),
            pltpu.VMEM((CHUNK,), jnp.int32),
            pltpu.VMEM((CHUNK,), jnp.int32),
            pltpu.VMEM((CHUNK,), jnp.float32),
            pltpu.VMEM((CHUNK,), jnp.float32),
            pltpu.SemaphoreType.DMA,
            pltpu.SemaphoreType.DMA,
            pltpu.SemaphoreType.DMA,
            pltpu.SemaphoreType.DMA,
            pltpu.SemaphoreType.DMA,
            pltpu.SemaphoreType.DMA,
            pltpu.SemaphoreType.DMA,
        ],
    )
    out = run(idx, upd, zeros)
    return out.reshape(B, HO, WO, CC)
